# baseline (speedup 1.0000x reference)
import jax
import jax.numpy as jnp
from jax.experimental import pallas as pl
from jax.experimental.pallas import tpu as pltpu

_K_SPARSITY = 64
_R_BLK = 256
_C_BLK = 512


def _topk_mask_kernel(x_ref, w_ref, out_ref):
    j = pl.program_id(1)
    nc = pl.num_programs(1)
    s_blk = jax.lax.dot_general(
        x_ref[...], w_ref[...],
        dimension_numbers=(((1,), (1,)), ((), ())),
        preferred_element_type=jnp.float32,
    )
    off = pl.multiple_of(j * _C_BLK, _C_BLK)
    out_ref[:, pl.ds(off, _C_BLK)] = s_blk

    @pl.when(j == nc - 1)
    def _epilogue():
        bits = jax.lax.bitcast_convert_type(out_ref[...], jnp.int32)
        key = jnp.where(bits < 0, bits ^ jnp.int32(0x7FFFFFFF), bits)
        out_ref[...] = jax.lax.bitcast_convert_type(key, jnp.float32)
        t = jnp.full((out_ref.shape[0], 1), jnp.iinfo(jnp.int32).min,
                     jnp.int32)
        for b in range(31, -1, -1):
            add = jnp.int32(-(2**31)) if b == 31 else jnp.int32(1 << b)
            cand = t + add
            key = jax.lax.bitcast_convert_type(out_ref[...], jnp.int32)
            cnt = jnp.sum((key >= cand).astype(jnp.int32), axis=1,
                          keepdims=True)
            t = jnp.where(cnt >= _K_SPARSITY, cand, t)
        key = jax.lax.bitcast_convert_type(out_ref[...], jnp.int32)
        s = jax.lax.bitcast_convert_type(
            jnp.where(key < 0, key ^ jnp.int32(0x7FFFFFFF), key),
            jnp.float32)
        mask = (key >= t) & (s > 0)
        out_ref[...] = jnp.where(mask, s, 0.0)


def kernel(x, W):
    B, K = x.shape
    N, K2 = W.shape
    assert K == K2 and B % _R_BLK == 0 and N % _C_BLK == 0
    grid = (B // _R_BLK, N // _C_BLK)
    return pl.pallas_call(
        _topk_mask_kernel,
        grid=grid,
        in_specs=[
            pl.BlockSpec((_R_BLK, K), lambda i, j: (i, 0)),
            pl.BlockSpec((_C_BLK, K), lambda i, j: (j, 0)),
        ],
        out_specs=pl.BlockSpec((_R_BLK, N), lambda i, j: (i, 0)),
        out_shape=jax.ShapeDtypeStruct((B, N), jnp.float32),
        compiler_params=pltpu.CompilerParams(
            dimension_semantics=("arbitrary", "arbitrary"),
        ),
    )(x, W)


# speedup vs baseline: 1.0214x; 1.0214x over previous
import jax
import jax.numpy as jnp
from jax.experimental import pallas as pl
from jax.experimental.pallas import tpu as pltpu

_K_SPARSITY = 64
_R_BLK = 256
_C_BLK = 256


def kernel(x, W):
    B, K = x.shape
    N, K2 = W.shape
    assert K == K2 and B % _R_BLK == 0 and N % _C_BLK == 0
    n_row = B // _R_BLK
    nc = N // _C_BLK
    bits_per_step = -(-32 // nc)

    def body(x_ref, w_ref, out_ref, acc0, acc1, t_ref):
        i = pl.program_id(0)
        j = pl.program_id(1)

        def do_matmul(acc):
            s_blk = jax.lax.dot_general(
                x_ref[...], w_ref[...],
                dimension_numbers=(((1,), (1,)), ((), ())),
                preferred_element_type=jnp.float32,
            )
            off = pl.multiple_of(j * _C_BLK, _C_BLK)
            acc[:, pl.ds(off, _C_BLK)] = s_blk

        def do_search(acc):
            @pl.when(j == 0)
            def _to_keys():
                bits = jax.lax.bitcast_convert_type(acc[...], jnp.int32)
                key = jnp.where(bits < 0, bits ^ jnp.int32(0x7FFFFFFF), bits)
                acc[...] = jax.lax.bitcast_convert_type(key, jnp.float32)
                t_ref[...] = jnp.full(t_ref.shape, jnp.iinfo(jnp.int32).min,
                                      jnp.int32)

            for db in range(bits_per_step):
                shift = jnp.maximum(31 - (bits_per_step * j + db), 0)
                add = jnp.left_shift(jnp.int32(1), shift)
                key = jax.lax.bitcast_convert_type(acc[...], jnp.int32)
                cand = t_ref[...] + add
                cnt = jnp.sum((key >= cand).astype(jnp.int32), axis=1,
                              keepdims=True)
                t_ref[...] = jnp.where(cnt >= _K_SPARSITY, cand, t_ref[...])

            @pl.when(j == nc - 1)
            def _mask_write():
                key = jax.lax.bitcast_convert_type(acc[...], jnp.int32)
                s = jax.lax.bitcast_convert_type(
                    jnp.where(key < 0, key ^ jnp.int32(0x7FFFFFFF), key),
                    jnp.float32)
                mask = (key >= t_ref[...]) & (s > 0)
                out_ref[...] = jnp.where(mask, s, 0.0)

        par = jax.lax.rem(i, 2)

        @pl.when((par == 0) & (i < n_row))
        def _mm0():
            do_matmul(acc0)

        @pl.when((par == 1) & (i < n_row))
        def _mm1():
            do_matmul(acc1)

        @pl.when((par == 1) & (i > 0))
        def _se0():
            do_search(acc0)

        @pl.when((par == 0) & (i > 0))
        def _se1():
            do_search(acc1)

    return pl.pallas_call(
        body,
        grid=(n_row + 1, nc),
        in_specs=[
            pl.BlockSpec((_R_BLK, K),
                         lambda i, j: (jnp.minimum(i, n_row - 1), 0)),
            pl.BlockSpec((_C_BLK, K), lambda i, j: (j, 0)),
        ],
        out_specs=pl.BlockSpec((_R_BLK, N),
                               lambda i, j: (jnp.maximum(i - 1, 0), 0)),
        out_shape=jax.ShapeDtypeStruct((B, N), jnp.float32),
        scratch_shapes=[
            pltpu.VMEM((_R_BLK, N), jnp.float32),
            pltpu.VMEM((_R_BLK, N), jnp.float32),
            pltpu.VMEM((_R_BLK, 1), jnp.int32),
        ],
        compiler_params=pltpu.CompilerParams(
            dimension_semantics=("arbitrary", "arbitrary"),
        ),
    )(x, W)
